# baseline (speedup 1.0000x reference)
import functools

import jax
import jax.numpy as jnp
from jax import lax
from jax.experimental import pallas as pl
from jax.experimental.pallas import tpu as pltpu
from jax.experimental.pallas import tpu_sc as plsc

VOC = 1000
H = 512
E = 256
B = 128
T = 50
G4 = 4 * H
NW = 32
ROWS = B * T
RPW = ROWS // NW


def _lstm_update(g, c):
    ii = jax.nn.sigmoid(g[:, 0 * H:1 * H])
    ff = jax.nn.sigmoid(g[:, 1 * H:2 * H])
    gg = jnp.tanh(g[:, 2 * H:3 * H])
    oo = jax.nn.sigmoid(g[:, 3 * H:4 * H])
    c2 = ff * c + ii * gg
    return oo * jnp.tanh(c2), c2


def _dot(a, w):
    return jnp.dot(a.astype(jnp.bfloat16), w, preferred_element_type=jnp.float32)


def _fused_body(x0_ref, wi0_ref, wh0_ref, b0_ref,
                wi1_ref, wh1_ref, b1_ref,
                wi2_ref, wh2_ref, b2_ref,
                wl_ref, bl_ref,
                out_ref, hn_ref, cn_ref,
                h0s, c0s, h1s, c1s, h2s, c2s):
    w = pl.program_id(0)

    @pl.when(w == 0)
    def _init():
        z = jnp.zeros((B, H), jnp.float32)
        h0s[...], c0s[...], h1s[...] = z, z, z
        c1s[...], h2s[...], c2s[...] = z, z, z

    h0, c0 = h0s[...], c0s[...]
    h1, c1 = h1s[...], c1s[...]
    h2, c2 = h2s[...], c2s[...]
    hb0 = h0.astype(jnp.bfloat16)
    hb1 = h1.astype(jnp.bfloat16)

    x = x0_ref[...].reshape(B, E)
    g0 = _dot(x, wi0_ref[...]) + _dot(h0, wh0_ref[...]) + b0_ref[...]
    g1 = jnp.dot(hb0, wi1_ref[...], preferred_element_type=jnp.float32) \
        + _dot(h1, wh1_ref[...]) + b1_ref[...]
    g2 = jnp.dot(hb1, wi2_ref[...], preferred_element_type=jnp.float32) \
        + _dot(h2, wh2_ref[...]) + b2_ref[...]
    nh0, nc0 = _lstm_update(g0, c0)
    nh1, nc1 = _lstm_update(g1, c1)
    nh2, nc2 = _lstm_update(g2, c2)

    out_ref[...] = (_dot(nh2, wl_ref[...])
                    + bl_ref[...]).reshape(1, B, VOC)

    @pl.when(w < T)
    def _u0():
        h0s[...], c0s[...] = nh0, nc0

    @pl.when(jnp.logical_and(w >= 1, w <= T))
    def _u1():
        h1s[...], c1s[...] = nh1, nc1

    @pl.when(w >= 2)
    def _u2():
        h2s[...], c2s[...] = nh2, nc2

    @pl.when(w == T + 1)
    def _fin():
        hn_ref[0], cn_ref[0] = h0, c0
        hn_ref[1], cn_ref[1] = h1, c1
        hn_ref[2], cn_ref[2] = nh2, nc2


def _fused_call(x0, wi0, wh0, b0, wi1, wh1, b1, wi2, wh2, b2, wl, bl):
    full = lambda w: (0, 0)
    full3 = lambda w: (0, 0, 0)
    return pl.pallas_call(
        _fused_body,
        grid=(T + 2,),
        in_specs=[
            pl.BlockSpec((1, B, E), lambda w: (jnp.minimum(w, T - 1), 0, 0)),
            pl.BlockSpec((E, G4), full), pl.BlockSpec((H, G4), full),
            pl.BlockSpec((1, G4), full),
            pl.BlockSpec((H, G4), full), pl.BlockSpec((H, G4), full),
            pl.BlockSpec((1, G4), full),
            pl.BlockSpec((H, G4), full), pl.BlockSpec((H, G4), full),
            pl.BlockSpec((1, G4), full),
            pl.BlockSpec((H, VOC), full), pl.BlockSpec((1, VOC), full),
        ],
        out_specs=[
            pl.BlockSpec((1, B, VOC), lambda w: (jnp.maximum(w - 2, 0), 0, 0)),
            pl.BlockSpec((3, B, H), full3),
            pl.BlockSpec((3, B, H), full3),
        ],
        out_shape=[
            jax.ShapeDtypeStruct((T, B, VOC), jnp.float32),
            jax.ShapeDtypeStruct((3, B, H), jnp.float32),
            jax.ShapeDtypeStruct((3, B, H), jnp.float32),
        ],
        scratch_shapes=[pltpu.VMEM((B, H), jnp.float32) for _ in range(6)],
    )(x0, wi0, wh0, b0, wi1, wh1, b1, wi2, wh2, b2, wl, bl)


def _gather_body(table_hbm, idx_hbm, out_hbm, idx_v, rows_v, sem):
    wid = lax.axis_index("s") * 2 + lax.axis_index("c")
    base = wid * RPW
    pltpu.sync_copy(idx_hbm.at[pl.ds(base, RPW)], idx_v)
    pltpu.async_copy(table_hbm.at[idx_v], rows_v, sem).wait()
    pltpu.sync_copy(rows_v, out_hbm.at[pl.ds(base, RPW)])


def _sc_gather(table, idx):
    mesh = plsc.VectorSubcoreMesh(core_axis_name="c", subcore_axis_name="s")
    k = functools.partial(
        pl.kernel, mesh=mesh,
        out_type=jax.ShapeDtypeStruct((ROWS, E // 2), jnp.int32),
        scratch_types=[
            pltpu.VMEM((RPW,), jnp.int32),
            pltpu.VMEM((RPW, E // 2), jnp.int32),
            pltpu.SemaphoreType.DMA,
        ],
    )(_gather_body)
    return k(table, idx)


def kernel(input_vector, embedding,
           W_ih_0, W_hh_0, b_ih_0, b_hh_0,
           W_ih_1, W_hh_1, b_ih_1, b_hh_1,
           W_ih_2, W_hh_2, b_ih_2, b_hh_2,
           W_lin, b_lin):
    bf = jnp.bfloat16
    idx_tm = input_vector.T.reshape(-1)
    table_i32 = jax.lax.bitcast_convert_type(
        embedding.astype(bf).reshape(VOC, E // 2, 2), jnp.int32)
    x0 = jax.lax.bitcast_convert_type(
        _sc_gather(table_i32, idx_tm), bf).reshape(T, B, E)

    wi0, wh0 = W_ih_0.T.astype(bf), W_hh_0.T.astype(bf)
    wi1, wh1 = W_ih_1.T.astype(bf), W_hh_1.T.astype(bf)
    wi2, wh2 = W_ih_2.T.astype(bf), W_hh_2.T.astype(bf)
    b0 = (b_ih_0 + b_hh_0).reshape(1, G4)
    b1 = (b_ih_1 + b_hh_1).reshape(1, G4)
    b2 = (b_ih_2 + b_hh_2).reshape(1, G4)
    wl = W_lin.T.astype(bf)
    bl = b_lin.reshape(1, VOC)

    out_tm, h_n, c_n = _fused_call(x0, wi0, wh0, b0, wi1, wh1, b1,
                                   wi2, wh2, b2, wl, bl)
    output_data = out_tm.reshape(B, T, VOC)
    return output_data, h_n, c_n


# speedup vs baseline: 1.0970x; 1.0970x over previous
import functools

import jax
import jax.numpy as jnp
from jax import lax
from jax.experimental import pallas as pl
from jax.experimental.pallas import tpu as pltpu
from jax.experimental.pallas import tpu_sc as plsc

VOC = 1000
H = 512
E = 256
B = 128
T = 50
G4 = 4 * H
NW = 32
ROWS = B * T
RPW = ROWS // NW


def _lstm_update(g, c):
    ii = jax.nn.sigmoid(g[:, 0 * H:1 * H])
    ff = jax.nn.sigmoid(g[:, 1 * H:2 * H])
    gg = jnp.tanh(g[:, 2 * H:3 * H])
    oo = jax.nn.sigmoid(g[:, 3 * H:4 * H])
    c2 = ff * c + ii * gg
    return oo * jnp.tanh(c2), c2


def _dot(a, w):
    return jnp.dot(a.astype(jnp.bfloat16), w, preferred_element_type=jnp.float32)


def _fused_body(x0_ref, wi0_ref, wh0_ref, b0_ref,
                wi1_ref, wh1_ref, b1_ref,
                wi2_ref, wh2_ref, b2_ref,
                wl_ref, bl_ref,
                out_ref, hn_ref, cn_ref,
                h0s, c0s, h1s, c1s, h2s, c2s):
    w = pl.program_id(0)

    @pl.when(w == 0)
    def _init():
        z = jnp.zeros((B, H), jnp.float32)
        h0s[...], c0s[...], h1s[...] = z, z, z
        c1s[...], h2s[...], c2s[...] = z, z, z

    h0, c0 = h0s[...], c0s[...]
    h1, c1 = h1s[...], c1s[...]
    h2, c2 = h2s[...], c2s[...]
    hb0 = h0.astype(jnp.bfloat16)
    hb1 = h1.astype(jnp.bfloat16)

    x = x0_ref[...].reshape(B, E)
    g0 = _dot(x, wi0_ref[...]) + _dot(h0, wh0_ref[...]) + b0_ref[...]
    g1 = jnp.dot(hb0, wi1_ref[...], preferred_element_type=jnp.float32) \
        + _dot(h1, wh1_ref[...]) + b1_ref[...]
    g2 = jnp.dot(hb1, wi2_ref[...], preferred_element_type=jnp.float32) \
        + _dot(h2, wh2_ref[...]) + b2_ref[...]
    nh0, nc0 = _lstm_update(g0, c0)
    nh1, nc1 = _lstm_update(g1, c1)
    nh2, nc2 = _lstm_update(g2, c2)

    out_ref[...] = (_dot(nh2, wl_ref[...])
                    + bl_ref[...]).reshape(1, B, VOC)

    @pl.when(w < T)
    def _u0():
        h0s[...], c0s[...] = nh0, nc0

    @pl.when(jnp.logical_and(w >= 1, w <= T))
    def _u1():
        h1s[...], c1s[...] = nh1, nc1

    @pl.when(w >= 2)
    def _u2():
        h2s[...], c2s[...] = nh2, nc2

    @pl.when(w == T + 1)
    def _fin():
        hn_ref[0], cn_ref[0] = h0, c0
        hn_ref[1], cn_ref[1] = h1, c1
        hn_ref[2], cn_ref[2] = nh2, nc2


def _fused_call(x0, wi0, wh0, b0, wi1, wh1, b1, wi2, wh2, b2, wl, bl):
    full = lambda w: (0, 0)
    full3 = lambda w: (0, 0, 0)
    return pl.pallas_call(
        _fused_body,
        grid=(T + 2,),
        in_specs=[
            pl.BlockSpec((1, B, E), lambda w: (jnp.minimum(w, T - 1), 0, 0)),
            pl.BlockSpec((E, G4), full), pl.BlockSpec((H, G4), full),
            pl.BlockSpec((1, G4), full),
            pl.BlockSpec((H, G4), full), pl.BlockSpec((H, G4), full),
            pl.BlockSpec((1, G4), full),
            pl.BlockSpec((H, G4), full), pl.BlockSpec((H, G4), full),
            pl.BlockSpec((1, G4), full),
            pl.BlockSpec((H, VOC), full), pl.BlockSpec((1, VOC), full),
        ],
        out_specs=[
            pl.BlockSpec((1, B, VOC), lambda w: (jnp.maximum(w - 2, 0), 0, 0)),
            pl.BlockSpec((3, B, H), full3),
            pl.BlockSpec((3, B, H), full3),
        ],
        out_shape=[
            jax.ShapeDtypeStruct((T, B, VOC), jnp.float32),
            jax.ShapeDtypeStruct((3, B, H), jnp.float32),
            jax.ShapeDtypeStruct((3, B, H), jnp.float32),
        ],
        scratch_shapes=[pltpu.VMEM((B, H), jnp.float32) for _ in range(6)],
    )(x0, wi0, wh0, b0, wi1, wh1, b1, wi2, wh2, b2, wl, bl)


def _gather_body(table_hbm, idx_hbm, out_hbm, idx_v, rows_v, sem):
    wid = lax.axis_index("s") * 2 + lax.axis_index("c")
    base = wid * RPW
    pltpu.sync_copy(idx_hbm.at[pl.ds(base, RPW)], idx_v)
    pltpu.async_copy(table_hbm.at[idx_v], rows_v, sem).wait()
    pltpu.sync_copy(rows_v, out_hbm.at[pl.ds(base, RPW)])


def _sc_gather(table, idx):
    mesh = plsc.VectorSubcoreMesh(core_axis_name="c", subcore_axis_name="s")
    k = functools.partial(
        pl.kernel, mesh=mesh,
        out_type=jax.ShapeDtypeStruct((ROWS, E // 2), jnp.int32),
        scratch_types=[
            pltpu.VMEM((RPW,), jnp.int32),
            pltpu.VMEM((RPW, E // 2), jnp.int32),
            pltpu.SemaphoreType.DMA,
        ],
    )(_gather_body)
    return k(table, idx)


def kernel(input_vector, embedding,
           W_ih_0, W_hh_0, b_ih_0, b_hh_0,
           W_ih_1, W_hh_1, b_ih_1, b_hh_1,
           W_ih_2, W_hh_2, b_ih_2, b_hh_2,
           W_lin, b_lin):
    bf = jnp.bfloat16
    idx_tm = input_vector.T.reshape(-1)
    table_i32 = jax.lax.bitcast_convert_type(
        embedding.astype(bf).reshape(VOC, E // 2, 2), jnp.int32)
    x0 = jax.lax.bitcast_convert_type(
        _sc_gather(table_i32, idx_tm), bf).reshape(T, B, E)

    wi0, wh0 = W_ih_0.T.astype(bf), W_hh_0.T.astype(bf)
    wi1, wh1 = W_ih_1.T.astype(bf), W_hh_1.T.astype(bf)
    wi2, wh2 = W_ih_2.T.astype(bf), W_hh_2.T.astype(bf)
    b0 = (b_ih_0 + b_hh_0).reshape(1, G4)
    b1 = (b_ih_1 + b_hh_1).reshape(1, G4)
    b2 = (b_ih_2 + b_hh_2).reshape(1, G4)
    wl = W_lin.T.astype(bf)
    bl = b_lin.reshape(1, VOC)

    out_tm, h_n, c_n = _fused_call(x0, wi0, wh0, b0, wi1, wh1, b1,
                                   wi2, wh2, b2, wl, bl)
    output_data = out_tm.transpose(1, 0, 2)
    return output_data, h_n, c_n


# speedup vs baseline: 1.1241x; 1.0247x over previous
import functools

import jax
import jax.numpy as jnp
from jax import lax
from jax.experimental import pallas as pl
from jax.experimental.pallas import tpu as pltpu
from jax.experimental.pallas import tpu_sc as plsc

VOC = 1000
H = 512
E = 256
B = 128
T = 50
G4 = 4 * H
NW = 32
ROWS = B * T
RPW = ROWS // NW


def _lstm_update(g, c):
    ii = jax.nn.sigmoid(g[:, 0 * H:1 * H])
    ff = jax.nn.sigmoid(g[:, 1 * H:2 * H])
    gg = jnp.tanh(g[:, 2 * H:3 * H])
    oo = jax.nn.sigmoid(g[:, 3 * H:4 * H])
    c2 = ff * c + ii * gg
    return oo * jnp.tanh(c2), c2


def _dot(a, w):
    return jnp.dot(a.astype(jnp.bfloat16), w, preferred_element_type=jnp.float32)


def _fused_body(x0_ref, wi0_ref, wh0_ref, b0_ref,
                wi1_ref, wh1_ref, b1_ref,
                wi2_ref, wh2_ref, b2_ref,
                wl_ref, bl_ref,
                out_ref, hn_ref, cn_ref,
                h0s, c0s, h1s, c1s, h2s, c2s):
    w = pl.program_id(0)

    @pl.when(w == 0)
    def _init():
        z = jnp.zeros((B, H), jnp.float32)
        h0s[...], c0s[...], h1s[...] = z, z, z
        c1s[...], h2s[...], c2s[...] = z, z, z

    h0, c0 = h0s[...], c0s[...]
    h1, c1 = h1s[...], c1s[...]
    h2, c2 = h2s[...], c2s[...]
    hb0 = h0.astype(jnp.bfloat16)
    hb1 = h1.astype(jnp.bfloat16)

    out_ref[...] = (_dot(h2, wl_ref[...])
                    + bl_ref[...]).reshape(1, B, VOC)

    x = x0_ref[...].reshape(B, E)
    g0 = _dot(x, wi0_ref[...]) + _dot(h0, wh0_ref[...]) + b0_ref[...]
    g1 = jnp.dot(hb0, wi1_ref[...], preferred_element_type=jnp.float32) \
        + _dot(h1, wh1_ref[...]) + b1_ref[...]
    g2 = jnp.dot(hb1, wi2_ref[...], preferred_element_type=jnp.float32) \
        + _dot(h2, wh2_ref[...]) + b2_ref[...]
    nh0, nc0 = _lstm_update(g0, c0)
    nh1, nc1 = _lstm_update(g1, c1)
    nh2, nc2 = _lstm_update(g2, c2)

    @pl.when(w < T)
    def _u0():
        h0s[...], c0s[...] = nh0, nc0

    @pl.when(jnp.logical_and(w >= 1, w <= T))
    def _u1():
        h1s[...], c1s[...] = nh1, nc1

    @pl.when(jnp.logical_and(w >= 2, w <= T + 1))
    def _u2():
        h2s[...], c2s[...] = nh2, nc2

    @pl.when(w == T + 2)
    def _fin():
        hn_ref[0], cn_ref[0] = h0, c0
        hn_ref[1], cn_ref[1] = h1, c1
        hn_ref[2], cn_ref[2] = h2, c2


def _fused_call(x0, wi0, wh0, b0, wi1, wh1, b1, wi2, wh2, b2, wl, bl):
    full = lambda w: (0, 0)
    full3 = lambda w: (0, 0, 0)
    return pl.pallas_call(
        _fused_body,
        grid=(T + 3,),
        in_specs=[
            pl.BlockSpec((1, B, E), lambda w: (jnp.minimum(w, T - 1), 0, 0)),
            pl.BlockSpec((E, G4), full), pl.BlockSpec((H, G4), full),
            pl.BlockSpec((1, G4), full),
            pl.BlockSpec((H, G4), full), pl.BlockSpec((H, G4), full),
            pl.BlockSpec((1, G4), full),
            pl.BlockSpec((H, G4), full), pl.BlockSpec((H, G4), full),
            pl.BlockSpec((1, G4), full),
            pl.BlockSpec((H, VOC), full), pl.BlockSpec((1, VOC), full),
        ],
        out_specs=[
            pl.BlockSpec((1, B, VOC), lambda w: (jnp.maximum(w - 3, 0), 0, 0)),
            pl.BlockSpec((3, B, H), full3),
            pl.BlockSpec((3, B, H), full3),
        ],
        out_shape=[
            jax.ShapeDtypeStruct((T, B, VOC), jnp.float32),
            jax.ShapeDtypeStruct((3, B, H), jnp.float32),
            jax.ShapeDtypeStruct((3, B, H), jnp.float32),
        ],
        scratch_shapes=[pltpu.VMEM((B, H), jnp.float32) for _ in range(6)],
    )(x0, wi0, wh0, b0, wi1, wh1, b1, wi2, wh2, b2, wl, bl)


def _gather_body(table_hbm, idx_hbm, out_hbm, idx_v, rows_v, sem):
    wid = lax.axis_index("s") * 2 + lax.axis_index("c")
    base = wid * RPW
    pltpu.sync_copy(idx_hbm.at[pl.ds(base, RPW)], idx_v)
    pltpu.async_copy(table_hbm.at[idx_v], rows_v, sem).wait()
    pltpu.sync_copy(rows_v, out_hbm.at[pl.ds(base, RPW)])


def _sc_gather(table, idx):
    mesh = plsc.VectorSubcoreMesh(core_axis_name="c", subcore_axis_name="s")
    k = functools.partial(
        pl.kernel, mesh=mesh,
        out_type=jax.ShapeDtypeStruct((ROWS, E // 2), jnp.int32),
        scratch_types=[
            pltpu.VMEM((RPW,), jnp.int32),
            pltpu.VMEM((RPW, E // 2), jnp.int32),
            pltpu.SemaphoreType.DMA,
        ],
    )(_gather_body)
    return k(table, idx)


def kernel(input_vector, embedding,
           W_ih_0, W_hh_0, b_ih_0, b_hh_0,
           W_ih_1, W_hh_1, b_ih_1, b_hh_1,
           W_ih_2, W_hh_2, b_ih_2, b_hh_2,
           W_lin, b_lin):
    bf = jnp.bfloat16
    idx_tm = input_vector.T.reshape(-1)
    table_i32 = jax.lax.bitcast_convert_type(
        embedding.astype(bf).reshape(VOC, E // 2, 2), jnp.int32)
    x0 = jax.lax.bitcast_convert_type(
        _sc_gather(table_i32, idx_tm), bf).reshape(T, B, E)

    wi0, wh0 = W_ih_0.T.astype(bf), W_hh_0.T.astype(bf)
    wi1, wh1 = W_ih_1.T.astype(bf), W_hh_1.T.astype(bf)
    wi2, wh2 = W_ih_2.T.astype(bf), W_hh_2.T.astype(bf)
    b0 = (b_ih_0 + b_hh_0).reshape(1, G4)
    b1 = (b_ih_1 + b_hh_1).reshape(1, G4)
    b2 = (b_ih_2 + b_hh_2).reshape(1, G4)
    wl = W_lin.T.astype(bf)
    bl = b_lin.reshape(1, VOC)

    out_tm, h_n, c_n = _fused_call(x0, wi0, wh0, b0, wi1, wh1, b1,
                                   wi2, wh2, b2, wl, bl)
    output_data = out_tm.transpose(1, 0, 2)
    return output_data, h_n, c_n


# speedup vs baseline: 1.1263x; 1.0019x over previous
import functools

import jax
import jax.numpy as jnp
from jax import lax
from jax.experimental import pallas as pl
from jax.experimental.pallas import tpu as pltpu
from jax.experimental.pallas import tpu_sc as plsc

VOC = 1000
H = 512
E = 256
B = 128
T = 50
G4 = 4 * H
NW = 32
ROWS = B * T
RPW = ROWS // NW


def _lstm_update(g, c):
    ii = jax.nn.sigmoid(g[:, 0 * H:1 * H])
    ff = jax.nn.sigmoid(g[:, 1 * H:2 * H])
    gg = jnp.tanh(g[:, 2 * H:3 * H])
    oo = jax.nn.sigmoid(g[:, 3 * H:4 * H])
    c2 = ff * c + ii * gg
    return oo * jnp.tanh(c2), c2


def _dot(a, w):
    return jnp.dot(a.astype(jnp.bfloat16), w, preferred_element_type=jnp.float32)


def _fused_body(x0_ref, wi0_ref, wh0_ref, b0_ref,
                wi1_ref, wh1_ref, b1_ref,
                wi2_ref, wh2_ref, b2_ref,
                wl_ref, bl_ref,
                out_ref, hn_ref, cn_ref,
                h0s, c0s, h1s, c1s, h2s, c2s):
    w = pl.program_id(0)

    @pl.when(w == 0)
    def _init():
        z = jnp.zeros((B, H), jnp.float32)
        zb = jnp.zeros((B, H), jnp.bfloat16)
        h0s[...], c0s[...], h1s[...] = zb, z, zb
        c1s[...], h2s[...], c2s[...] = z, zb, z

    h0, c0 = h0s[...], c0s[...]
    h1, c1 = h1s[...], c1s[...]
    h2, c2 = h2s[...], c2s[...]
    hb0 = h0
    hb1 = h1

    out_ref[...] = (_dot(h2, wl_ref[...])
                    + bl_ref[...]).reshape(1, B, VOC)

    x = x0_ref[...].reshape(B, E)
    g0 = _dot(x, wi0_ref[...]) + _dot(h0, wh0_ref[...]) + b0_ref[...]
    g1 = jnp.dot(hb0, wi1_ref[...], preferred_element_type=jnp.float32) \
        + _dot(h1, wh1_ref[...]) + b1_ref[...]
    g2 = jnp.dot(hb1, wi2_ref[...], preferred_element_type=jnp.float32) \
        + _dot(h2, wh2_ref[...]) + b2_ref[...]
    nh0, nc0 = _lstm_update(g0, c0)
    nh1, nc1 = _lstm_update(g1, c1)
    nh2, nc2 = _lstm_update(g2, c2)

    @pl.when(w < T)
    def _u0():
        h0s[...], c0s[...] = nh0.astype(jnp.bfloat16), nc0

    @pl.when(jnp.logical_and(w >= 1, w <= T))
    def _u1():
        h1s[...], c1s[...] = nh1.astype(jnp.bfloat16), nc1

    @pl.when(jnp.logical_and(w >= 2, w <= T + 1))
    def _u2():
        h2s[...], c2s[...] = nh2.astype(jnp.bfloat16), nc2

    @pl.when(w == T + 2)
    def _fin():
        hn_ref[0], cn_ref[0] = h0.astype(jnp.float32), c0
        hn_ref[1], cn_ref[1] = h1.astype(jnp.float32), c1
        hn_ref[2], cn_ref[2] = h2.astype(jnp.float32), c2


def _fused_call(x0, wi0, wh0, b0, wi1, wh1, b1, wi2, wh2, b2, wl, bl):
    full = lambda w: (0, 0)
    full3 = lambda w: (0, 0, 0)
    return pl.pallas_call(
        _fused_body,
        grid=(T + 3,),
        in_specs=[
            pl.BlockSpec((1, B, E), lambda w: (jnp.minimum(w, T - 1), 0, 0)),
            pl.BlockSpec((E, G4), full), pl.BlockSpec((H, G4), full),
            pl.BlockSpec((1, G4), full),
            pl.BlockSpec((H, G4), full), pl.BlockSpec((H, G4), full),
            pl.BlockSpec((1, G4), full),
            pl.BlockSpec((H, G4), full), pl.BlockSpec((H, G4), full),
            pl.BlockSpec((1, G4), full),
            pl.BlockSpec((H, VOC), full), pl.BlockSpec((1, VOC), full),
        ],
        out_specs=[
            pl.BlockSpec((1, B, VOC), lambda w: (jnp.maximum(w - 3, 0), 0, 0)),
            pl.BlockSpec((3, B, H), full3),
            pl.BlockSpec((3, B, H), full3),
        ],
        out_shape=[
            jax.ShapeDtypeStruct((T, B, VOC), jnp.float32),
            jax.ShapeDtypeStruct((3, B, H), jnp.float32),
            jax.ShapeDtypeStruct((3, B, H), jnp.float32),
        ],
        scratch_shapes=[pltpu.VMEM((B, H), dt) for dt in
                        (jnp.bfloat16, jnp.float32, jnp.bfloat16,
                         jnp.float32, jnp.bfloat16, jnp.float32)],
    )(x0, wi0, wh0, b0, wi1, wh1, b1, wi2, wh2, b2, wl, bl)


def _gather_body(table_hbm, idx_hbm, out_hbm, idx_v, rows_v, sem):
    wid = lax.axis_index("s") * 2 + lax.axis_index("c")
    base = wid * RPW
    pltpu.sync_copy(idx_hbm.at[pl.ds(base, RPW)], idx_v)
    pltpu.async_copy(table_hbm.at[idx_v], rows_v, sem).wait()
    pltpu.sync_copy(rows_v, out_hbm.at[pl.ds(base, RPW)])


def _sc_gather(table, idx):
    mesh = plsc.VectorSubcoreMesh(core_axis_name="c", subcore_axis_name="s")
    k = functools.partial(
        pl.kernel, mesh=mesh,
        out_type=jax.ShapeDtypeStruct((ROWS, E // 2), jnp.int32),
        scratch_types=[
            pltpu.VMEM((RPW,), jnp.int32),
            pltpu.VMEM((RPW, E // 2), jnp.int32),
            pltpu.SemaphoreType.DMA,
        ],
    )(_gather_body)
    return k(table, idx)


def kernel(input_vector, embedding,
           W_ih_0, W_hh_0, b_ih_0, b_hh_0,
           W_ih_1, W_hh_1, b_ih_1, b_hh_1,
           W_ih_2, W_hh_2, b_ih_2, b_hh_2,
           W_lin, b_lin):
    bf = jnp.bfloat16
    idx_tm = input_vector.T.reshape(-1)
    table_i32 = jax.lax.bitcast_convert_type(
        embedding.astype(bf).reshape(VOC, E // 2, 2), jnp.int32)
    x0 = jax.lax.bitcast_convert_type(
        _sc_gather(table_i32, idx_tm), bf).reshape(T, B, E)

    wi0, wh0 = W_ih_0.T.astype(bf), W_hh_0.T.astype(bf)
    wi1, wh1 = W_ih_1.T.astype(bf), W_hh_1.T.astype(bf)
    wi2, wh2 = W_ih_2.T.astype(bf), W_hh_2.T.astype(bf)
    b0 = (b_ih_0 + b_hh_0).reshape(1, G4)
    b1 = (b_ih_1 + b_hh_1).reshape(1, G4)
    b2 = (b_ih_2 + b_hh_2).reshape(1, G4)
    wl = W_lin.T.astype(bf)
    bl = b_lin.reshape(1, VOC)

    out_tm, h_n, c_n = _fused_call(x0, wi0, wh0, b0, wi1, wh1, b1,
                                   wi2, wh2, b2, wl, bl)
    output_data = out_tm.transpose(1, 0, 2)
    return output_data, h_n, c_n
